# baseline (speedup 1.0000x reference)
import jax
import jax.numpy as jnp
from jax import lax
from jax.experimental import pallas as pl
from jax.experimental.pallas import tpu as pltpu

N = 2048
D = 768
E = 8
K = 2


def _moe_kernel(x_ref, wg_ref, bg_ref, we_ref, be_ref, out_ref, scale_ref):
    e = pl.program_id(0)

    @pl.when(e == 0)
    def _():
        logits = lax.dot_general(
            x_ref[...], wg_ref[...], (((1,), (1,)), ((), ())),
            preferred_element_type=jnp.float32,
        ) + bg_ref[...]
        idx = lax.broadcasted_iota(jnp.int32, logits.shape, 1)
        m1 = jnp.max(logits, axis=1, keepdims=True)
        i1 = jnp.min(jnp.where(logits == m1, idx, E), axis=1, keepdims=True)
        oh1 = idx == i1
        masked = jnp.where(oh1, -jnp.inf, logits)
        m2 = jnp.max(masked, axis=1, keepdims=True)
        i2 = jnp.min(jnp.where(masked == m2, idx, E), axis=1, keepdims=True)
        oh2 = idx == i2
        cnt = jnp.sum(oh1.astype(jnp.float32) + oh2.astype(jnp.float32), axis=0)
        scale_ref[...] = (cnt / float(N * K)).reshape(1, E)

    w = we_ref[0]
    z = lax.dot_general(
        x_ref[...], w, (((1,), (1,)), ((), ())),
        preferred_element_type=jnp.float32,
    )
    r = jnp.maximum(z + be_ref[0], 0.0)
    sel = lax.broadcasted_iota(jnp.int32, (1, E), 1) == e
    s = jnp.sum(jnp.where(sel, scale_ref[...], 0.0), axis=(0, 1), keepdims=True)
    contrib = r * s

    @pl.when(e == 0)
    def _():
        out_ref[...] = contrib

    @pl.when(e > 0)
    def _():
        out_ref[...] += contrib


def kernel(x, Wg, bg, We, be):
    out = pl.pallas_call(
        _moe_kernel,
        grid=(E,),
        in_specs=[
            pl.BlockSpec((N, D), lambda e: (0, 0)),
            pl.BlockSpec((E, D), lambda e: (0, 0)),
            pl.BlockSpec((1, E), lambda e: (0, 0)),
            pl.BlockSpec((1, D, D), lambda e: (e, 0, 0)),
            pl.BlockSpec((1, 1, D), lambda e: (e, 0, 0)),
        ],
        out_specs=pl.BlockSpec((N, D), lambda e: (0, 0)),
        out_shape=jax.ShapeDtypeStruct((N, D), jnp.float32),
        scratch_shapes=[pltpu.VMEM((1, E), jnp.float32)],
    )(
        x.astype(jnp.bfloat16),
        Wg.astype(jnp.bfloat16),
        bg.reshape(1, E),
        We.astype(jnp.bfloat16),
        be.reshape(E, 1, D),
    )
    return out


# speedup vs baseline: 1.3700x; 1.3700x over previous
import jax
import jax.numpy as jnp
from jax import lax
from jax.experimental import pallas as pl
from jax.experimental.pallas import tpu as pltpu

N = 2048
D = 768
E = 8
K = 2


def _moe_kernel(x_ref, wg_ref, bg_ref, we_ref, be_ref, out_ref, scale_ref, xb_ref):
    e = pl.program_id(0)

    @pl.when(e == 0)
    def _():
        xb_ref[...] = x_ref[...].astype(jnp.bfloat16)
        logits = lax.dot_general(
            x_ref[...], wg_ref[...], (((1,), (1,)), ((), ())),
            preferred_element_type=jnp.float32,
        ) + bg_ref[...]
        idx = lax.broadcasted_iota(jnp.int32, logits.shape, 1)
        m1 = jnp.max(logits, axis=1, keepdims=True)
        i1 = jnp.min(jnp.where(logits == m1, idx, E), axis=1, keepdims=True)
        oh1 = idx == i1
        masked = jnp.where(oh1, -jnp.inf, logits)
        m2 = jnp.max(masked, axis=1, keepdims=True)
        i2 = jnp.min(jnp.where(masked == m2, idx, E), axis=1, keepdims=True)
        oh2 = idx == i2
        cnt = jnp.sum(oh1.astype(jnp.float32) + oh2.astype(jnp.float32), axis=0)
        scale_ref[...] = (cnt / float(N * K)).reshape(1, E)

    w = we_ref[0].astype(jnp.bfloat16)
    z = lax.dot_general(
        xb_ref[...], w, (((1,), (1,)), ((), ())),
        preferred_element_type=jnp.float32,
    )
    r = jnp.maximum(z + be_ref[0], 0.0)
    sel = lax.broadcasted_iota(jnp.int32, (1, E), 1) == e
    s = jnp.sum(jnp.where(sel, scale_ref[...], 0.0), axis=(0, 1), keepdims=True)
    contrib = r * s

    @pl.when(e == 0)
    def _():
        out_ref[...] = contrib

    @pl.when(e > 0)
    def _():
        out_ref[...] += contrib


def kernel(x, Wg, bg, We, be):
    out = pl.pallas_call(
        _moe_kernel,
        grid=(E,),
        in_specs=[
            pl.BlockSpec((N, D), lambda e: (0, 0)),
            pl.BlockSpec((E, D), lambda e: (0, 0)),
            pl.BlockSpec((1, E), lambda e: (0, 0)),
            pl.BlockSpec((1, D, D), lambda e: (e, 0, 0)),
            pl.BlockSpec((1, 1, D), lambda e: (e, 0, 0)),
        ],
        out_specs=pl.BlockSpec((N, D), lambda e: (0, 0)),
        out_shape=jax.ShapeDtypeStruct((N, D), jnp.float32),
        scratch_shapes=[
            pltpu.VMEM((1, E), jnp.float32),
            pltpu.VMEM((N, D), jnp.bfloat16),
        ],
    )(x, Wg, bg.reshape(1, E), We, be.reshape(E, 1, D))
    return out
